# baseline (speedup 1.0000x reference)
import functools
import jax
import jax.numpy as jnp
from jax import lax
from jax.experimental import pallas as pl
from jax.experimental.pallas import tpu as pltpu
from jax.experimental.pallas import tpu_sc as plsc

L = 16


def _round_bf16(x):
    return x.astype(jnp.bfloat16).astype(jnp.float32)


def _tc_stage1_body(flow_ref, s_ref):
    n = flow_ref.shape[1]
    xr = _round_bf16(flow_ref[...])
    mu = jnp.sum(xr, axis=1, keepdims=True) / jnp.float32(n)
    s_ref[...] = jnp.sum(xr * mu, axis=2)[:, None, :]


def _tc_stage2_body(sup_ref, km_ref, suprgb_ref, krgb_ref, out_ref):
    m = sup_ref.shape[1]
    xr = _round_bf16(sup_ref[...])
    km = km_ref[...]
    s = jnp.sum(xr * km, axis=2)[0]
    iota = lax.iota(jnp.int32, m)
    big = jnp.int32(2 ** 30)
    m1 = jnp.max(s)
    i1 = jnp.min(jnp.where(s == m1, iota, big))
    s2 = jnp.where(iota == i1, -jnp.inf, s)
    m2 = jnp.max(s2)
    i2 = jnp.min(jnp.where(s2 == m2, iota, big))
    rgb = suprgb_ref[...]
    iota3 = lax.broadcasted_iota(jnp.int32, (1, m, 1), 1)
    r1 = jnp.sum(jnp.where(iota3 == i1, rgb, 0.0),
                 axis=1, keepdims=True)
    r2 = jnp.sum(jnp.where(iota3 == i2, rgb, 0.0),
                 axis=1, keepdims=True)
    out_ref[...] = jnp.concatenate([krgb_ref[...], r1, r2], axis=1)


def _stage1_scores(flow):
    b, n, f = flow.shape
    s = pl.pallas_call(
        _tc_stage1_body,
        grid=(b,),
        in_specs=[pl.BlockSpec((1, n, f), lambda i: (i, 0, 0))],
        out_specs=pl.BlockSpec((1, 1, n), lambda i: (i, 0, 0)),
        out_shape=jax.ShapeDtypeStruct((b, 1, n), jnp.float32),
    )(flow)
    return s.reshape(b, n)


def _stage2_select(sup_flow, kmean, sup_rgb, krgb):
    b, m, f = sup_flow.shape
    return pl.pallas_call(
        _tc_stage2_body,
        grid=(b,),
        in_specs=[
            pl.BlockSpec((1, m, f), lambda i: (i, 0, 0)),
            pl.BlockSpec((1, 1, f), lambda i: (i, 0, 0)),
            pl.BlockSpec((1, m, f), lambda i: (i, 0, 0)),
            pl.BlockSpec((1, 3, f), lambda i: (i, 0, 0)),
        ],
        out_specs=pl.BlockSpec((1, 5, f), lambda i: (i, 0, 0)),
        out_shape=jax.ShapeDtypeStruct((b, 5, f), jnp.float32),
    )(sup_flow, kmean.reshape(b, 1, f), sup_rgb, krgb)


def _sc_round_bf16(x):
    w = plsc.bitcast(x, jnp.int32)
    tie = lax.shift_right_logical(w, 16) & jnp.int32(1)
    wr = (w + jnp.int32(0x7FFF) + tie) & jnp.int32(-65536)
    return plsc.bitcast(wr, jnp.float32)


def _lane_iota():
    return lax.iota(jnp.int32, 16)


def _scan_topk(s_v, n, k, largest):
    groups = n // L
    sentinel = jnp.float32(-jnp.inf) if largest else jnp.float32(jnp.inf)
    iota = _lane_iota()

    def better(a, bv):
        return a > bv if largest else a < bv

    def body(g, carry):
        m1, i1, m2, i2, m3, i3 = carry
        s = s_v[pl.ds(g * L, L)]
        nvec = g * L + iota
        c1 = better(s, m1)
        c2 = better(s, m2)
        c3 = better(s, m3)
        nm3 = jnp.where(c2, m2, jnp.where(c3, s, m3))
        ni3 = jnp.where(c2, i2, jnp.where(c3, nvec, i3))
        nm2 = jnp.where(c1, m1, jnp.where(c2, s, m2))
        ni2 = jnp.where(c1, i1, jnp.where(c2, nvec, i2))
        nm1 = jnp.where(c1, s, m1)
        ni1 = jnp.where(c1, nvec, i1)
        return nm1, ni1, nm2, ni2, nm3, ni3

    full = jnp.full((L,), sentinel, jnp.float32)
    zero = jnp.zeros((L,), jnp.int32)
    m1, i1, m2, i2, m3, i3 = lax.fori_loop(
        0, groups, body, (full, zero, full, zero, full, zero))

    idx_g = jnp.zeros((L,), jnp.int32)
    big = jnp.int32(2 ** 30)
    for r in range(k):
        gbest = jnp.max(m1) if largest else jnp.min(m1)
        isel = jnp.min(jnp.where(m1 == gbest, i1, big))
        upd = (m1 == gbest) & (i1 == isel)
        idx_g = jnp.where(iota == r, isel, idx_g)
        m1 = jnp.where(upd, m2, m1)
        i1 = jnp.where(upd, i2, i1)
        m2 = jnp.where(upd, m3, m2)
        i2 = jnp.where(upd, i3, i2)
        m3 = jnp.where(upd, jnp.full((L,), sentinel, jnp.float32), m3)
    return idx_g


def _make_sc1(b, n, f):
    mesh = plsc.VectorSubcoreMesh(core_axis_name="c", subcore_axis_name="s")

    @functools.partial(
        pl.kernel,
        mesh=mesh,
        out_type=(
            jax.ShapeDtypeStruct((b, 3, f), jnp.float32),
            jax.ShapeDtypeStruct((b, f), jnp.float32),
        ),
        compiler_params=pltpu.CompilerParams(needs_layout_passes=False, use_tc_tiling_on_sc=False),
        scratch_types=[
            pltpu.VMEM((n,), jnp.float32),
            pltpu.VMEM((L, f), jnp.float32),
            pltpu.VMEM((L, f), jnp.float32),
            pltpu.VMEM((2 * L,), jnp.float32),
            pltpu.SemaphoreType.DMA,
            pltpu.SemaphoreType.DMA,
        ],
    )
    def sc1(s_hbm, rgb_hbm, flow_hbm, krgb_hbm, km_hbm, s_v, rows_rgb,
            rows_flow, km_v, sem1, sem2):
        cid = lax.axis_index("c")
        sid = lax.axis_index("s")

        @pl.when(cid == 0)
        def _():
            batch = sid
            pltpu.sync_copy(s_hbm.at[batch], s_v)
            idx_g = _scan_topk(s_v, n, 3, largest=False)
            cp1 = pltpu.async_copy(rgb_hbm.at[batch].at[idx_g], rows_rgb, sem1)
            cp2 = pltpu.async_copy(flow_hbm.at[batch].at[idx_g], rows_flow, sem2)
            cp1.wait()
            cp2.wait()
            for h in range(2):
                acc = jnp.zeros((L,), jnp.float32)
                for j in range(3):
                    acc = acc + _sc_round_bf16(rows_flow[j, pl.ds(h * L, L)])
                km_v[pl.ds(h * L, L)] = acc / jnp.float32(3.0)
            pltpu.sync_copy(rows_rgb.at[pl.ds(0, 3)], krgb_hbm.at[batch])
            pltpu.sync_copy(km_v, km_hbm.at[batch])

    return sc1


def kernel(ref_rgb_feat, ref_flow_feat, sup_rgb_feat, sup_flow_feat):
    b, n, f = ref_flow_feat.shape
    m = sup_flow_feat.shape[1]

    s1 = _stage1_scores(ref_flow_feat)
    krgb, kmean = _make_sc1(b, n, f)(s1, ref_rgb_feat, ref_flow_feat)
    return _stage2_select(sup_flow_feat, kmean, sup_rgb_feat, krgb)


# speedup vs baseline: 1.0866x; 1.0866x over previous
import functools
import jax
import jax.numpy as jnp
from jax import lax
from jax.experimental import pallas as pl
from jax.experimental.pallas import tpu as pltpu
from jax.experimental.pallas import tpu_sc as plsc

L = 16


def _round_bf16(x):
    return x.astype(jnp.bfloat16).astype(jnp.float32)


def _group_fold(prod, f):
    g = (lax.broadcasted_iota(jnp.int32, (4 * f, 4), 0) // f
         == lax.broadcasted_iota(jnp.int32, (4 * f, 4), 1)).astype(jnp.float32)
    return lax.dot_general(prod, g, (((1,), (0,)), ((), ())),
                           preferred_element_type=jnp.float32)


def _fold_lanes(v, f):
    h = (lax.broadcasted_iota(jnp.int32, (4 * f, f), 0) % f
         == lax.broadcasted_iota(jnp.int32, (4 * f, f), 1)).astype(jnp.float32)
    return lax.dot_general(v, h, (((1,), (0,)), ((), ())),
                           preferred_element_type=jnp.float32)


def _tc_stage1_body(flow_ref, s_ref):
    r = flow_ref.shape[1]
    f = flow_ref.shape[2] // 4
    xr = _round_bf16(flow_ref[...])
    mu128 = jnp.sum(xr, axis=1)
    mu32 = _fold_lanes(mu128, f) / jnp.float32(4 * r)
    mut = jnp.concatenate([mu32] * 4, axis=1)[None]
    s4 = _group_fold((xr * mut)[0], f)
    s_ref[...] = s4[None]


def _tc_stage2_body(sup_ref, km_ref, suprgb_ref, krgb_ref, out_ref):
    r = sup_ref.shape[1]
    f = km_ref.shape[2]
    xr = _round_bf16(sup_ref[...])
    kmt = jnp.concatenate([km_ref[...]] * 4, axis=2)
    s4 = _group_fold((xr * kmt)[0], f)
    iota_n = (lax.broadcasted_iota(jnp.int32, (r, 4), 0) * 4
              + lax.broadcasted_iota(jnp.int32, (r, 4), 1))
    big = jnp.int32(2 ** 30)
    m1 = jnp.max(s4)
    i1 = jnp.min(jnp.where(s4 == m1, iota_n, big))
    s2 = jnp.where(iota_n == i1, -jnp.inf, s4)
    m2 = jnp.max(s2)
    i2 = jnp.min(jnp.where(s2 == m2, iota_n, big))
    rgb = suprgb_ref[...]
    iota_fl = (lax.broadcasted_iota(jnp.int32, (1, r, 4 * f), 1) * 4
               + lax.broadcasted_iota(jnp.int32, (1, r, 4 * f), 2) // f)

    def pick(i):
        sel = jnp.sum(jnp.where(iota_fl == i, rgb, 0.0), axis=1)
        return _fold_lanes(sel, f)[None]

    out_ref[...] = jnp.concatenate([krgb_ref[...], pick(i1), pick(i2)], axis=1)


def _stage1_scores(flow):
    b, n, f = flow.shape
    r = n // 4
    s = pl.pallas_call(
        _tc_stage1_body,
        grid=(b,),
        in_specs=[pl.BlockSpec((1, r, 4 * f), lambda i: (i, 0, 0))],
        out_specs=pl.BlockSpec((1, r, 4), lambda i: (i, 0, 0)),
        out_shape=jax.ShapeDtypeStruct((b, r, 4), jnp.float32),
    )(flow.reshape(b, r, 4 * f))
    return s.reshape(b, n)


def _stage2_select(sup_flow, kmean, sup_rgb, krgb):
    b, m, f = sup_flow.shape
    r = m // 4
    return pl.pallas_call(
        _tc_stage2_body,
        grid=(b,),
        in_specs=[
            pl.BlockSpec((1, r, 4 * f), lambda i: (i, 0, 0)),
            pl.BlockSpec((1, 1, f), lambda i: (i, 0, 0)),
            pl.BlockSpec((1, r, 4 * f), lambda i: (i, 0, 0)),
            pl.BlockSpec((1, 3, f), lambda i: (i, 0, 0)),
        ],
        out_specs=pl.BlockSpec((1, 5, f), lambda i: (i, 0, 0)),
        out_shape=jax.ShapeDtypeStruct((b, 5, f), jnp.float32),
    )(sup_flow.reshape(b, r, 4 * f), kmean.reshape(b, 1, f),
      sup_rgb.reshape(b, r, 4 * f), krgb)


def _sc_round_bf16(x):
    w = plsc.bitcast(x, jnp.int32)
    tie = lax.shift_right_logical(w, 16) & jnp.int32(1)
    wr = (w + jnp.int32(0x7FFF) + tie) & jnp.int32(-65536)
    return plsc.bitcast(wr, jnp.float32)


def _lane_iota():
    return lax.iota(jnp.int32, 16)


def _scan_topk(s_v, n, k, largest):
    groups = n // L
    sentinel = jnp.float32(-jnp.inf) if largest else jnp.float32(jnp.inf)
    iota = _lane_iota()

    def better(a, bv):
        return a > bv if largest else a < bv

    def body(g, carry):
        m1, i1, m2, i2, m3, i3 = carry
        s = s_v[pl.ds(g * L, L)]
        nvec = g * L + iota
        c1 = better(s, m1)
        c2 = better(s, m2)
        c3 = better(s, m3)
        nm3 = jnp.where(c2, m2, jnp.where(c3, s, m3))
        ni3 = jnp.where(c2, i2, jnp.where(c3, nvec, i3))
        nm2 = jnp.where(c1, m1, jnp.where(c2, s, m2))
        ni2 = jnp.where(c1, i1, jnp.where(c2, nvec, i2))
        nm1 = jnp.where(c1, s, m1)
        ni1 = jnp.where(c1, nvec, i1)
        return nm1, ni1, nm2, ni2, nm3, ni3

    full = jnp.full((L,), sentinel, jnp.float32)
    zero = jnp.zeros((L,), jnp.int32)
    m1, i1, m2, i2, m3, i3 = lax.fori_loop(
        0, groups, body, (full, zero, full, zero, full, zero))

    idx_g = jnp.zeros((L,), jnp.int32)
    big = jnp.int32(2 ** 30)
    for r in range(k):
        gbest = jnp.max(m1) if largest else jnp.min(m1)
        isel = jnp.min(jnp.where(m1 == gbest, i1, big))
        upd = (m1 == gbest) & (i1 == isel)
        idx_g = jnp.where(iota == r, isel, idx_g)
        m1 = jnp.where(upd, m2, m1)
        i1 = jnp.where(upd, i2, i1)
        m2 = jnp.where(upd, m3, m2)
        i2 = jnp.where(upd, i3, i2)
        m3 = jnp.where(upd, jnp.full((L,), sentinel, jnp.float32), m3)
    return idx_g


def _make_sc1(b, n, f):
    mesh = plsc.VectorSubcoreMesh(core_axis_name="c", subcore_axis_name="s")

    @functools.partial(
        pl.kernel,
        mesh=mesh,
        out_type=(
            jax.ShapeDtypeStruct((b, 3, f), jnp.float32),
            jax.ShapeDtypeStruct((b, f), jnp.float32),
        ),
        compiler_params=pltpu.CompilerParams(needs_layout_passes=False, use_tc_tiling_on_sc=False),
        scratch_types=[
            pltpu.VMEM((n,), jnp.float32),
            pltpu.VMEM((L, f), jnp.float32),
            pltpu.VMEM((L, f), jnp.float32),
            pltpu.VMEM((2 * L,), jnp.float32),
            pltpu.SemaphoreType.DMA,
            pltpu.SemaphoreType.DMA,
        ],
    )
    def sc1(s_hbm, rgb_hbm, flow_hbm, krgb_hbm, km_hbm, s_v, rows_rgb,
            rows_flow, km_v, sem1, sem2):
        cid = lax.axis_index("c")
        sid = lax.axis_index("s")

        @pl.when(cid == 0)
        def _():
            batch = sid
            pltpu.sync_copy(s_hbm.at[batch], s_v)
            idx_g = _scan_topk(s_v, n, 3, largest=False)
            cp1 = pltpu.async_copy(rgb_hbm.at[batch].at[idx_g], rows_rgb, sem1)
            cp2 = pltpu.async_copy(flow_hbm.at[batch].at[idx_g], rows_flow, sem2)
            cp1.wait()
            cp2.wait()
            for h in range(2):
                acc = jnp.zeros((L,), jnp.float32)
                for j in range(3):
                    acc = acc + _sc_round_bf16(rows_flow[j, pl.ds(h * L, L)])
                km_v[pl.ds(h * L, L)] = acc / jnp.float32(3.0)
            pltpu.sync_copy(rows_rgb.at[pl.ds(0, 3)], krgb_hbm.at[batch])
            pltpu.sync_copy(km_v, km_hbm.at[batch])

    return sc1


def kernel(ref_rgb_feat, ref_flow_feat, sup_rgb_feat, sup_flow_feat):
    b, n, f = ref_flow_feat.shape
    m = sup_flow_feat.shape[1]

    s1 = _stage1_scores(ref_flow_feat)
    krgb, kmean = _make_sc1(b, n, f)(s1, ref_rgb_feat, ref_flow_feat)
    return _stage2_select(sup_flow_feat, kmean, sup_rgb_feat, krgb)
